# Initial kernel scaffold; baseline (speedup 1.0000x reference)
import jax
import jax.numpy as jnp
from jax.experimental import pallas as pl


def kernel(x, W):
    raise NotImplementedError("write your pallas kernel here")


# speedup vs baseline: 1.3068x; 1.3068x over previous
import functools

import jax
import jax.numpy as jnp
from jax import lax
from jax.experimental import pallas as pl
from jax.experimental.pallas import tpu as pltpu
from jax.experimental.pallas import tpu_sc as plsc

NC = 2
NS = 16
NW = NC * NS
CH = 128


def _make_gather(N, V, D):
    b_per_w = N // NW
    n_ch = b_per_w // CH
    mesh = plsc.VectorSubcoreMesh(core_axis_name="c", subcore_axis_name="s")

    @functools.partial(
        pl.kernel,
        mesh=mesh,
        out_type=jax.ShapeDtypeStruct((NW, n_ch, CH, D), jnp.float32),
        scratch_types=[
            pltpu.VMEM((n_ch, CH), jnp.int32),
            pltpu.VMEM((CH, D), jnp.float32),
            pltpu.SemaphoreType.DMA,
        ],
        compiler_params=pltpu.CompilerParams(use_tc_tiling_on_sc=False),
    )
    def k(idx_hbm, table_hbm, out_hbm, idx_v, rows_v, sem):
        wid = lax.axis_index("s") * NC + lax.axis_index("c")
        pltpu.sync_copy(idx_hbm.at[wid], idx_v)

        def body(j, carry):
            pltpu.async_copy(table_hbm.at[idx_v.at[j]], rows_v, sem).wait()
            pltpu.sync_copy(rows_v, out_hbm.at[wid, j])
            return carry

        lax.fori_loop(0, n_ch, body, 0, unroll=False)

    return k


def kernel(x, W):
    B, H = x.shape
    V, D = W.shape
    N = B * H
    xf = x.astype(jnp.int32).reshape(NW, (N // NW) // CH, CH)
    out = _make_gather(N, V, D)(xf, W)
    return out.reshape(B, H, D)


# speedup vs baseline: 1.4933x; 1.1427x over previous
import functools

import jax
import jax.numpy as jnp
from jax import lax
from jax.experimental import pallas as pl
from jax.experimental.pallas import tpu as pltpu
from jax.experimental.pallas import tpu_sc as plsc

NC = 2
NS = 16
NW = NC * NS
CH = 128
K = 10


def _make_gather(N, V, D):
    b_per_w = N // NW
    n_ch = b_per_w // CH
    n_sch = n_ch // K
    n_half = n_sch // 2
    SCR = K * CH
    assert n_sch % 2 == 0 and n_sch * K == n_ch
    mesh = plsc.VectorSubcoreMesh(core_axis_name="c", subcore_axis_name="s")

    @functools.partial(
        pl.kernel,
        mesh=mesh,
        out_type=jax.ShapeDtypeStruct((NW, b_per_w, D), jnp.float32),
        scratch_types=[
            pltpu.VMEM((n_ch, CH), jnp.int32),
            pltpu.VMEM((2, SCR, D), jnp.float32),
            pltpu.SemaphoreType.DMA,
            pltpu.SemaphoreType.DMA,
        ],
        compiler_params=pltpu.CompilerParams(use_tc_tiling_on_sc=False),
    )
    def k(idx_hbm, table_hbm, out_hbm, idx_v, buf, sem_g, sem_s):
        wid = lax.axis_index("s") * NC + lax.axis_index("c")
        pltpu.sync_copy(idx_hbm.at[wid], idx_v)

        def fire(s, b):
            for t in range(K):
                pltpu.async_copy(
                    table_hbm.at[idx_v.at[s * K + t]],
                    buf.at[b, pl.ds(t * CH, CH)],
                    sem_g,
                )

        def wait_gathers(b):
            pltpu.make_async_copy(
                table_hbm.at[pl.ds(0, SCR)], buf.at[b], sem_g
            ).wait()

        def store(s, b):
            pltpu.async_copy(
                buf.at[b], out_hbm.at[wid, pl.ds(s * SCR, SCR)], sem_s
            )

        def wait_store(s, b):
            pltpu.make_async_copy(
                buf.at[b], out_hbm.at[wid, pl.ds(s * SCR, SCR)], sem_s
            ).wait()

        fire(0, 0)

        def body(i, carry):
            s0 = i * 2
            wait_gathers(0)

            @pl.when(i > 0)
            def _():
                wait_store(s0 - 1, 1)

            fire(s0 + 1, 1)
            store(s0, 0)
            wait_gathers(1)

            @pl.when(i < n_half - 1)
            def _():
                wait_store(s0, 0)
                fire(s0 + 2, 0)

            store(s0 + 1, 1)
            return carry

        lax.fori_loop(0, n_half, body, 0, unroll=False)
        wait_store(n_sch - 2, 0)
        wait_store(n_sch - 1, 1)

    return k


def kernel(x, W):
    B, H = x.shape
    V, D = W.shape
    N = B * H
    xf = x.astype(jnp.int32).reshape(NW, (N // NW) // CH, CH)
    out = _make_gather(N, V, D)(xf, W)
    return out.reshape(B, H, D)
